# baseline (device time: 153999 ns/iter reference)
import jax
import jax.numpy as jnp
from jax import lax
from jax.experimental import pallas as pl
from jax.experimental.pallas import tpu as pltpu

try:
    jax.config.update("jax_compilation_cache_dir", "/tmp/jaxcache")
except Exception:
    pass

N_DEV = 32
E_LOC = 4
N_EXP = 128
D_MODEL = 256
D_HID = 512
N_TOK = 1024
C = 48
BLK = E_LOC * C
COLS = N_EXP * C


def kernel(x, router_W, route_idx, expert_W):
    assert x.shape == (N_TOK, D_MODEL), x.shape
    assert expert_W.shape == (E_LOC, D_MODEL, D_HID), expert_W.shape

    def body(x_ref, rw_ref, idx_ref, ew_ref, out_ref,
             disp_ref, recv_ref, ret_ref, retr_ref,
             dsend, drecv, rsend, rrecv):
        my = lax.axis_index("i")

        bsem = pltpu.get_barrier_semaphore()

        def bar(dd, carry):
            d = lax.rem(my + dd, N_DEV)
            pl.semaphore_signal(bsem, inc=1, device_id=(d,),
                                device_id_type=pl.DeviceIdType.MESH)
            return carry
        lax.fori_loop(1, N_DEV, bar, 0)
        pl.semaphore_wait(bsem, N_DEV - 1)

        xv = x_ref[...]
        xb = xv.astype(jnp.bfloat16)
        scores = jnp.dot(xv, rw_ref[...], preferred_element_type=jnp.float32)
        smax = jnp.max(scores, axis=1, keepdims=True)
        p = jnp.exp(scores - smax)
        p = p / jnp.sum(p, axis=1, keepdims=True)
        idx0 = idx_ref[:, 0:1]
        idx1 = idx_ref[:, 1:2]
        eids = lax.broadcasted_iota(jnp.int32, (N_TOK, N_EXP), 1)
        g0 = jnp.sum(jnp.where(eids == idx0, p, 0.0), axis=1, keepdims=True)
        g1 = jnp.sum(jnp.where(eids == idx1, p, 0.0), axis=1, keepdims=True)
        gs = g0 + g1
        w0 = g0 / gs
        w1 = g1 / gs

        eids_t = lax.broadcasted_iota(jnp.int32, (N_EXP, N_TOK), 0)
        idx0_t = idx0.reshape(1, N_TOK)
        idx1_t = idx1.reshape(1, N_TOK)
        cmp0 = eids_t == idx0_t
        cmp1 = eids_t == idx1_t
        pair = cmp0.astype(jnp.int32) + cmp1.astype(jnp.int32)
        cp = pair
        sh = 1
        while sh < N_TOK:
            cp = cp + jnp.concatenate(
                [jnp.zeros((N_EXP, sh), jnp.int32), cp[:, :-sh]], axis=1)
            sh *= 2
        cp = cp - pair

        kio = lax.broadcasted_iota(jnp.int32, (N_EXP, C, N_TOK), 1)
        at_k = cp[:, None, :] == kio
        hit0 = (cmp0[:, None, :] & at_k).astype(jnp.bfloat16)
        hit1 = (cmp1[:, None, :] & at_k).astype(jnp.bfloat16)
        u_t = (hit0 + hit1).reshape(COLS, N_TOK)
        w0_t = w0.astype(jnp.bfloat16).reshape(1, 1, N_TOK)
        w1_t = w1.astype(jnp.bfloat16).reshape(1, 1, N_TOK)
        sw_t = (hit0 * w0_t + hit1 * w1_t).reshape(COLS, N_TOK)

        disp = jnp.dot(u_t, xb, preferred_element_type=jnp.float32)
        disp_ref[...] = disp.astype(jnp.bfloat16).reshape(N_DEV, BLK, D_MODEL)

        def p1_send(dd, carry):
            d = lax.rem(my + dd, N_DEV)
            pltpu.make_async_remote_copy(
                src_ref=disp_ref.at[d], dst_ref=recv_ref.at[my],
                send_sem=dsend.at[dd], recv_sem=drecv.at[dd],
                device_id=(d,), device_id_type=pl.DeviceIdType.MESH,
            ).start()
            return carry
        lax.fori_loop(1, N_DEV, p1_send, 0)
        recv_ref[pl.ds(my, 1)] = disp_ref[pl.ds(my, 1)]

        def p1_wait(dd, carry):
            pltpu.make_async_remote_copy(
                src_ref=disp_ref.at[0], dst_ref=recv_ref.at[0],
                send_sem=dsend.at[dd], recv_sem=drecv.at[dd],
                device_id=(0,), device_id_type=pl.DeviceIdType.MESH,
            ).wait_recv()
            return carry
        lax.fori_loop(1, N_DEV, p1_wait, 0)

        recv = recv_ref[...]
        ew_b = ew_ref[...].astype(jnp.bfloat16)
        for e in range(E_LOC):
            rows = recv[:, e * C:(e + 1) * C, :].reshape(N_DEV * C, D_MODEL)
            res = jnp.dot(rows, ew_b[e], preferred_element_type=jnp.float32)
            ret_ref[:, e * C:(e + 1) * C, :] = (
                res.astype(jnp.bfloat16).reshape(N_DEV, C, D_HID))

        def p2_send(dd, carry):
            s = lax.rem(my + dd, N_DEV)
            pltpu.make_async_remote_copy(
                src_ref=ret_ref.at[s], dst_ref=retr_ref.at[my],
                send_sem=rsend.at[dd], recv_sem=rrecv.at[dd],
                device_id=(s,), device_id_type=pl.DeviceIdType.MESH,
            ).start()
            return carry
        lax.fori_loop(1, N_DEV, p2_send, 0)
        retr_ref[pl.ds(my, 1)] = ret_ref[pl.ds(my, 1)]

        def p2_wait(dd, carry):
            pltpu.make_async_remote_copy(
                src_ref=ret_ref.at[0], dst_ref=retr_ref.at[0],
                send_sem=rsend.at[dd], recv_sem=rrecv.at[dd],
                device_id=(0,), device_id_type=pl.DeviceIdType.MESH,
            ).wait_recv()
            return carry
        lax.fori_loop(1, N_DEV, p2_wait, 0)

        retr = retr_ref[...].reshape(COLS, D_HID)
        out_ref[...] = lax.dot_general(
            sw_t, retr, (((0,), (0,)), ((), ())),
            preferred_element_type=jnp.float32)

        def drain(dd, carry):
            pltpu.make_async_remote_copy(
                src_ref=disp_ref.at[0], dst_ref=recv_ref.at[0],
                send_sem=dsend.at[dd], recv_sem=drecv.at[dd],
                device_id=(0,), device_id_type=pl.DeviceIdType.MESH,
            ).wait_send()
            pltpu.make_async_remote_copy(
                src_ref=ret_ref.at[0], dst_ref=retr_ref.at[0],
                send_sem=rsend.at[dd], recv_sem=rrecv.at[dd],
                device_id=(0,), device_id_type=pl.DeviceIdType.MESH,
            ).wait_send()
            return carry
        lax.fori_loop(1, N_DEV, drain, 0)

    return pl.pallas_call(
        body,
        out_shape=jax.ShapeDtypeStruct((N_TOK, D_HID), jnp.float32),
        in_specs=[
            pl.BlockSpec(memory_space=pltpu.VMEM),
            pl.BlockSpec(memory_space=pltpu.VMEM),
            pl.BlockSpec(memory_space=pltpu.VMEM),
            pl.BlockSpec(memory_space=pltpu.VMEM),
        ],
        out_specs=pl.BlockSpec(memory_space=pltpu.VMEM),
        scratch_shapes=[
            pltpu.VMEM((N_DEV, BLK, D_MODEL), jnp.bfloat16),
            pltpu.VMEM((N_DEV, BLK, D_MODEL), jnp.bfloat16),
            pltpu.VMEM((N_DEV, BLK, D_HID), jnp.bfloat16),
            pltpu.VMEM((N_DEV, BLK, D_HID), jnp.bfloat16),
            pltpu.SemaphoreType.DMA((N_DEV,)),
            pltpu.SemaphoreType.DMA((N_DEV,)),
            pltpu.SemaphoreType.DMA((N_DEV,)),
            pltpu.SemaphoreType.DMA((N_DEV,)),
        ],
        compiler_params=pltpu.CompilerParams(
            collective_id=0, vmem_limit_bytes=110 * 1024 * 1024),
    )(x, router_W, route_idx, expert_W)
